# baseline (speedup 1.0000x reference)
import jax
import jax.numpy as jnp
from jax.experimental import pallas as pl
from jax.experimental.pallas import tpu as pltpu

BB = 16
MB = 128
F32 = jnp.float32
BF16 = jnp.bfloat16


def _conv_from(src, w_ref, b_ref, ho, wo, bb):
    cin = w_ref.shape[1]
    acc = None
    for t in range(9):
        ki, kj = divmod(t, 3)
        xs = src[:, ki:ki + ho, kj:kj + wo, :].reshape(bb * ho * wo, cin)
        p = jnp.dot(xs, w_ref[t], preferred_element_type=F32)
        acc = p if acc is None else acc + p
    acc = acc + b_ref[...]
    return jnp.maximum(acc, 0.0)


def _pool2x2(val, h, w, c, bb):
    v = val.reshape(bb * h * w // 2, 2, c)
    v = jnp.maximum(v[:, 0, :], v[:, 1, :])
    v = v.reshape(bb * h // 2, 2, w // 2, c)
    v = jnp.maximum(v[:, 0], v[:, 1])
    return v.reshape(bb * h * w // 4, c)


def _store_padded(dst, val, h, w, c, bb):
    z_row = jnp.zeros((bb, 1, w + 2, c), dtype=BF16)
    z_col = jnp.zeros((bb, h, 1, c), dtype=BF16)
    dst[:, 0:1, :, :] = z_row
    dst[:, h + 1:h + 2, :, :] = z_row
    dst[:, 1:h + 1, 0:1, :] = z_col
    dst[:, 1:h + 1, w + 1:w + 2, :] = z_col
    dst[:, 1:h + 1, 1:w + 1, :] = val.astype(BF16).reshape(bb, h, w, c)


def _convs_kernel(xc_ref, w1, b1, w2, b2, w3, b3, w4, b4, w5, b5,
                  o_ref, p2, p3, p4, p5):
    xs = xc_ref[...].reshape(BB * 32 * 32, 27)
    a = jnp.dot(xs, w1[...], preferred_element_type=F32) + b1[...]
    a = jnp.maximum(a, 0.0)
    _store_padded(p2, a, 32, 32, 32, BB)
    a = _conv_from(p2, w2, b2, 32, 32, BB)
    a = _pool2x2(a, 32, 32, 32, BB)
    _store_padded(p3, a, 16, 16, 32, BB)
    a = _conv_from(p3, w3, b3, 16, 16, BB)
    _store_padded(p4, a, 16, 16, 64, BB)
    a = _conv_from(p4, w4, b4, 16, 16, BB)
    a = _pool2x2(a, 16, 16, 128, BB)
    _store_padded(p5, a, 8, 8, 128, BB)
    a = _conv_from(p5, w5, b5, 8, 8, BB)
    o_ref[...] = a.astype(BF16).reshape(BB, 8, 8, 256)


def _fc_kernel(x_ref, w1, b1, w2, b2, w3, b3, w4, b4, o_ref):
    acc = None
    for h in range(6):
        for w in range(6):
            xs = x_ref[:, h + 1, w + 1, :]
            wblk = w1[(h * 6 + w) * 256:(h * 6 + w + 1) * 256, :]
            p = jnp.dot(xs, wblk, preferred_element_type=F32)
            acc = p if acc is None else acc + p
    h = jnp.maximum(acc + b1[...], 0.0).astype(BF16)
    h = jnp.dot(h, w2[...], preferred_element_type=F32) + b2[...]
    h = jnp.maximum(h, 0.0).astype(BF16)
    h = jnp.dot(h, w3[...], preferred_element_type=F32) + b3[...]
    h = jnp.maximum(h, 0.0).astype(BF16)
    h = jnp.dot(h, w4[...], preferred_element_type=F32) + b4[...]
    o_ref[...] = h


def kernel(conv1_w, conv1_b, conv2_w, conv2_b, conv3_w, conv3_b,
           conv4_w, conv4_b, conv5_w, conv5_b,
           fc1_w, fc1_b, fc2_w, fc2_b, fc3_w, fc3_b, fc4_w, fc4_b, x):
    B = x.shape[0]
    xp = jnp.pad(jnp.transpose(x, (0, 2, 3, 1)).astype(BF16),
                 ((0, 0), (1, 1), (1, 1), (0, 0)))
    xcol = jnp.concatenate(
        [xp[:, ki:ki + 32, kj:kj + 32, :] for ki in range(3) for kj in range(3)],
        axis=-1)

    w1c = conv1_w.reshape(27, 32).astype(BF16)
    cw = [w.astype(BF16) for w in (conv2_w, conv3_w, conv4_w, conv5_w)]
    cb = (conv2_b, conv3_b, conv4_b, conv5_b)

    def wspec(shape):
        return pl.BlockSpec(shape, lambda i: (0,) * len(shape))

    conv_in_specs = [pl.BlockSpec((BB, 32, 32, 27), lambda i: (i, 0, 0, 0)),
                     wspec(w1c.shape), wspec(conv1_b.shape)]
    for w, b in zip(cw, cb):
        conv_in_specs.append(wspec(w.shape))
        conv_in_specs.append(wspec(b.shape))

    feat = pl.pallas_call(
        _convs_kernel,
        out_shape=jax.ShapeDtypeStruct((B, 8, 8, 256), BF16),
        grid=(B // BB,),
        in_specs=conv_in_specs,
        out_specs=pl.BlockSpec((BB, 8, 8, 256), lambda i: (i, 0, 0, 0)),
        scratch_shapes=[
            pltpu.VMEM((BB, 34, 34, 32), BF16),
            pltpu.VMEM((BB, 18, 18, 32), BF16),
            pltpu.VMEM((BB, 18, 18, 64), BF16),
            pltpu.VMEM((BB, 10, 10, 128), BF16),
        ],
        compiler_params=pltpu.CompilerParams(
            dimension_semantics=("parallel",),
            vmem_limit_bytes=56 * 1024 * 1024),
    )(xcol, w1c, conv1_b, cw[0], cb[0], cw[1], cb[1], cw[2], cb[2], cw[3], cb[3])

    fw = [w.astype(BF16) for w in (fc1_w, fc2_w, fc3_w, fc4_w)]
    fb = (fc1_b, fc2_b, fc3_b, fc4_b)

    fc_in_specs = [pl.BlockSpec((MB, 8, 8, 256), lambda i: (i, 0, 0, 0))]
    for w, b in zip(fw, fb):
        fc_in_specs.append(wspec(w.shape))
        fc_in_specs.append(wspec(b.shape))

    out = pl.pallas_call(
        _fc_kernel,
        out_shape=jax.ShapeDtypeStruct((B, 2), F32),
        grid=(B // MB,),
        in_specs=fc_in_specs,
        out_specs=pl.BlockSpec((MB, 2), lambda i: (i, 0)),
        compiler_params=pltpu.CompilerParams(
            dimension_semantics=("parallel",),
            vmem_limit_bytes=56 * 1024 * 1024),
    )(feat, fw[0], fb[0], fw[1], fb[1], fw[2], fb[2], fw[3], fb[3])
    return out


# speedup vs baseline: 1.6771x; 1.6771x over previous
import jax
import jax.numpy as jnp
from jax.experimental import pallas as pl
from jax.experimental.pallas import tpu as pltpu

BB = 8
MB = 128
F32 = jnp.float32
BF16 = jnp.bfloat16


def _conv_x3(src, w_ref, b_ref, h, w, c, cout):
    hp = h + 2
    xs = src[...].reshape(BB * hp * w, 3 * c)
    y = jnp.dot(xs, w_ref[...], preferred_element_type=F32)
    y = y.reshape(BB, hp, w, 3 * cout)
    acc = (y[:, 0:h, :, 0:cout]
           + y[:, 1:h + 1, :, cout:2 * cout]
           + y[:, 2:h + 2, :, 2 * cout:3 * cout])
    acc = acc + b_ref[...]
    return jnp.maximum(acc, 0.0)


def _pool2x2(val, h, w, c, bb):
    v = val.reshape(bb * h * w // 2, 2, c)
    v = jnp.maximum(v[:, 0, :], v[:, 1, :])
    v = v.reshape(bb * h // 2, 2, w // 2, c)
    v = jnp.maximum(v[:, 0], v[:, 1])
    return v.reshape(bb, h // 2, w // 2, c)


def _x3_store(dst, val, h, w, c):
    a4 = val.astype(BF16)
    z_row = jnp.zeros((BB, 1, w, 3 * c), dtype=BF16)
    z_col = jnp.zeros((BB, h, 1, c), dtype=BF16)
    dst[:, 0:1, :, :] = z_row
    dst[:, h + 1:h + 2, :, :] = z_row
    dst[:, 1:h + 1, 1:w, 0:c] = a4[:, :, 0:w - 1, :]
    dst[:, 1:h + 1, 0:1, 0:c] = z_col
    dst[:, 1:h + 1, :, c:2 * c] = a4
    dst[:, 1:h + 1, 0:w - 1, 2 * c:3 * c] = a4[:, :, 1:w, :]
    dst[:, 1:h + 1, w - 1:w, 2 * c:3 * c] = z_col


def _convs_kernel(x3_ref, w1, b1, w2, b2, w3, b3, w4, b4, w5, b5,
                  o_ref, p2, p3, p4, p5):
    a = _conv_x3(x3_ref, w1, b1, 32, 32, 3, 32)
    _x3_store(p2, a, 32, 32, 32)
    a = _conv_x3(p2, w2, b2, 32, 32, 32, 32)
    a = _pool2x2(a, 32, 32, 32, BB)
    _x3_store(p3, a, 16, 16, 32)
    a = _conv_x3(p3, w3, b3, 16, 16, 32, 64)
    _x3_store(p4, a, 16, 16, 64)
    a = _conv_x3(p4, w4, b4, 16, 16, 64, 128)
    a = _pool2x2(a, 16, 16, 128, BB)
    _x3_store(p5, a, 8, 8, 128)
    a = _conv_x3(p5, w5, b5, 8, 8, 128, 256)
    o_ref[...] = a.astype(BF16)


def _fc_kernel(x_ref, w1, b1, w2, b2, w3, b3, w4, b4, o_ref):
    acc = None
    for h in range(6):
        for w in range(6):
            xs = x_ref[:, h + 1, w + 1, :]
            wblk = w1[(h * 6 + w) * 256:(h * 6 + w + 1) * 256, :]
            p = jnp.dot(xs, wblk, preferred_element_type=F32)
            acc = p if acc is None else acc + p
    h = jnp.maximum(acc + b1[...], 0.0).astype(BF16)
    h = jnp.dot(h, w2[...], preferred_element_type=F32) + b2[...]
    h = jnp.maximum(h, 0.0).astype(BF16)
    h = jnp.dot(h, w3[...], preferred_element_type=F32) + b3[...]
    h = jnp.maximum(h, 0.0).astype(BF16)
    h = jnp.dot(h, w4[...], preferred_element_type=F32) + b4[...]
    o_ref[...] = h


def _x3_weights(w_taps):
    _, c, o = w_taps.shape
    return (w_taps.reshape(3, 3, c, o).transpose(1, 2, 0, 3)
            .reshape(3 * c, 3 * o).astype(BF16))


def kernel(conv1_w, conv1_b, conv2_w, conv2_b, conv3_w, conv3_b,
           conv4_w, conv4_b, conv5_w, conv5_b,
           fc1_w, fc1_b, fc2_w, fc2_b, fc3_w, fc3_b, fc4_w, fc4_b, x):
    B = x.shape[0]
    xp = jnp.pad(jnp.transpose(x, (0, 2, 3, 1)).astype(BF16),
                 ((0, 0), (1, 1), (1, 1), (0, 0)))
    x3 = jnp.concatenate([xp[:, :, kj:kj + 32, :] for kj in range(3)],
                         axis=-1)

    cw = [_x3_weights(w) for w in (conv1_w, conv2_w, conv3_w, conv4_w, conv5_w)]
    cb = (conv1_b, conv2_b, conv3_b, conv4_b, conv5_b)

    def wspec(shape):
        return pl.BlockSpec(shape, lambda i: (0,) * len(shape))

    conv_in_specs = [pl.BlockSpec((BB, 34, 32, 9), lambda i: (i, 0, 0, 0))]
    for w, b in zip(cw, cb):
        conv_in_specs.append(wspec(w.shape))
        conv_in_specs.append(wspec(b.shape))

    feat = pl.pallas_call(
        _convs_kernel,
        out_shape=jax.ShapeDtypeStruct((B, 8, 8, 256), BF16),
        grid=(B // BB,),
        in_specs=conv_in_specs,
        out_specs=pl.BlockSpec((BB, 8, 8, 256), lambda i: (i, 0, 0, 0)),
        scratch_shapes=[
            pltpu.VMEM((BB, 34, 32, 96), BF16),
            pltpu.VMEM((BB, 18, 16, 96), BF16),
            pltpu.VMEM((BB, 18, 16, 192), BF16),
            pltpu.VMEM((BB, 10, 8, 384), BF16),
        ],
        compiler_params=pltpu.CompilerParams(
            dimension_semantics=("parallel",),
            vmem_limit_bytes=56 * 1024 * 1024),
    )(x3, cw[0], cb[0], cw[1], cb[1], cw[2], cb[2], cw[3], cb[3], cw[4], cb[4])

    fw = [w.astype(BF16) for w in (fc1_w, fc2_w, fc3_w, fc4_w)]
    fb = (fc1_b, fc2_b, fc3_b, fc4_b)

    fc_in_specs = [pl.BlockSpec((MB, 8, 8, 256), lambda i: (i, 0, 0, 0))]
    for w, b in zip(fw, fb):
        fc_in_specs.append(wspec(w.shape))
        fc_in_specs.append(wspec(b.shape))

    out = pl.pallas_call(
        _fc_kernel,
        out_shape=jax.ShapeDtypeStruct((B, 2), F32),
        grid=(B // MB,),
        in_specs=fc_in_specs,
        out_specs=pl.BlockSpec((MB, 2), lambda i: (i, 0)),
        compiler_params=pltpu.CompilerParams(
            dimension_semantics=("parallel",),
            vmem_limit_bytes=56 * 1024 * 1024),
    )(feat, fw[0], fb[0], fw[1], fb[1], fw[2], fb[2], fw[3], fb[3])
    return out


# speedup vs baseline: 2.3916x; 1.4260x over previous
import jax
import jax.numpy as jnp
from jax.experimental import pallas as pl
from jax.experimental.pallas import tpu as pltpu

BB = 16
MB = 128
F32 = jnp.float32
BF16 = jnp.bfloat16


def _conv_x3(src, w_ref, h, w, c, cout):
    hp = h + 2
    np_ = max(cout, 128)
    xs = src[...].reshape(BB * hp * w, 3 * c)
    y = jnp.dot(xs, w_ref[...], preferred_element_type=F32)
    y = y.reshape(BB, hp, w, 3 * np_)
    return (y[:, 0:h, :, 0:cout]
            + y[:, 1:h + 1, :, np_:np_ + cout]
            + y[:, 2:h + 2, :, 2 * np_:2 * np_ + cout])


def _bias_relu(val, b_ref):
    return jnp.maximum(val + b_ref[...], 0.0)


def _pool2x2(val, h, w, c, bb):
    v = val.reshape(bb * h * w // 2, 2, c)
    v = jnp.maximum(v[:, 0, :], v[:, 1, :])
    v = v.reshape(bb * h // 2, 2, w // 2, c)
    v = jnp.maximum(v[:, 0], v[:, 1])
    return v.reshape(bb, h // 2, w // 2, c)


def _x3_store(dst, val, h, w, c):
    z_col = jnp.zeros((BB, h, 1, c), dtype=F32)
    v0 = jnp.concatenate([z_col, val[:, :, 0:w - 1, :]], axis=2)
    v2 = jnp.concatenate([val[:, :, 1:w, :], z_col], axis=2)
    v3 = jnp.concatenate([v0, val, v2], axis=3).astype(BF16)
    z_row = jnp.zeros((BB, 1, w, 3 * c), dtype=BF16)
    dst[:, 0:1, :, :] = z_row
    dst[:, h + 1:h + 2, :, :] = z_row
    dst[:, 1:h + 1, :, :] = v3


def _convs_kernel(x3_ref, w1, b1, w2, b2, w3, b3, w4, b4, w5, b5,
                  o_ref, p2, p3, p4, p5):
    a = _bias_relu(_conv_x3(x3_ref, w1, 32, 32, 3, 32), b1)
    _x3_store(p2, a, 32, 32, 32)
    a = _conv_x3(p2, w2, 32, 32, 32, 32)
    a = _bias_relu(_pool2x2(a, 32, 32, 32, BB), b2)
    _x3_store(p3, a, 16, 16, 32)
    a = _bias_relu(_conv_x3(p3, w3, 16, 16, 32, 64), b3)
    _x3_store(p4, a, 16, 16, 64)
    a = _conv_x3(p4, w4, 16, 16, 64, 128)
    a = _bias_relu(_pool2x2(a, 16, 16, 128, BB), b4)
    _x3_store(p5, a, 8, 8, 128)
    a = _bias_relu(_conv_x3(p5, w5, 8, 8, 128, 256), b5)
    o_ref[...] = a.astype(BF16)


def _fc_kernel(x_ref, w1, b1, w2, b2, w3, b3, w4, b4, o_ref):
    acc = None
    for h in range(6):
        for w in range(6):
            xs = x_ref[:, h + 1, w + 1, :]
            wblk = w1[(h * 6 + w) * 256:(h * 6 + w + 1) * 256, :]
            p = jnp.dot(xs, wblk, preferred_element_type=F32)
            acc = p if acc is None else acc + p
    h = jnp.maximum(acc + b1[...], 0.0).astype(BF16)
    h = jnp.dot(h, w2[...], preferred_element_type=F32) + b2[...]
    h = jnp.maximum(h, 0.0).astype(BF16)
    h = jnp.dot(h, w3[...], preferred_element_type=F32) + b3[...]
    h = jnp.maximum(h, 0.0).astype(BF16)
    h = jnp.dot(h, w4[...], preferred_element_type=F32) + b4[...]
    o_ref[...] = h


def _x3_weights(w_taps):
    _, c, o = w_taps.shape
    wk = w_taps.reshape(3, 3, c, o).transpose(1, 2, 0, 3)
    np_ = max(o, 128)
    if np_ != o:
        wk = jnp.pad(wk, ((0, 0), (0, 0), (0, 0), (0, np_ - o)))
    return wk.reshape(3 * c, 3 * np_).astype(BF16)


def kernel(conv1_w, conv1_b, conv2_w, conv2_b, conv3_w, conv3_b,
           conv4_w, conv4_b, conv5_w, conv5_b,
           fc1_w, fc1_b, fc2_w, fc2_b, fc3_w, fc3_b, fc4_w, fc4_b, x):
    B = x.shape[0]
    xp = jnp.pad(jnp.transpose(x, (0, 2, 3, 1)).astype(BF16),
                 ((0, 0), (1, 1), (1, 1), (0, 0)))
    x3 = jnp.concatenate([xp[:, :, kj:kj + 32, :] for kj in range(3)],
                         axis=-1)

    cw = [_x3_weights(w) for w in (conv1_w, conv2_w, conv3_w, conv4_w, conv5_w)]
    cb = (conv1_b, conv2_b, conv3_b, conv4_b, conv5_b)

    def wspec(shape):
        return pl.BlockSpec(shape, lambda i: (0,) * len(shape))

    conv_in_specs = [pl.BlockSpec((BB, 34, 32, 9), lambda i: (i, 0, 0, 0))]
    for w, b in zip(cw, cb):
        conv_in_specs.append(wspec(w.shape))
        conv_in_specs.append(wspec(b.shape))

    feat = pl.pallas_call(
        _convs_kernel,
        out_shape=jax.ShapeDtypeStruct((B, 8, 8, 256), BF16),
        grid=(B // BB,),
        in_specs=conv_in_specs,
        out_specs=pl.BlockSpec((BB, 8, 8, 256), lambda i: (i, 0, 0, 0)),
        scratch_shapes=[
            pltpu.VMEM((BB, 34, 32, 96), BF16),
            pltpu.VMEM((BB, 18, 16, 96), BF16),
            pltpu.VMEM((BB, 18, 16, 192), BF16),
            pltpu.VMEM((BB, 10, 8, 384), BF16),
        ],
        compiler_params=pltpu.CompilerParams(
            dimension_semantics=("parallel",),
            vmem_limit_bytes=56 * 1024 * 1024),
    )(x3, cw[0], cb[0], cw[1], cb[1], cw[2], cb[2], cw[3], cb[3], cw[4], cb[4])

    fw = [w.astype(BF16) for w in (fc1_w, fc2_w, fc3_w, fc4_w)]
    fb = (fc1_b, fc2_b, fc3_b, fc4_b)

    fc_in_specs = [pl.BlockSpec((MB, 8, 8, 256), lambda i: (i, 0, 0, 0))]
    for w, b in zip(fw, fb):
        fc_in_specs.append(wspec(w.shape))
        fc_in_specs.append(wspec(b.shape))

    out = pl.pallas_call(
        _fc_kernel,
        out_shape=jax.ShapeDtypeStruct((B, 2), F32),
        grid=(B // MB,),
        in_specs=fc_in_specs,
        out_specs=pl.BlockSpec((MB, 2), lambda i: (i, 0)),
        compiler_params=pltpu.CompilerParams(
            dimension_semantics=("parallel",),
            vmem_limit_bytes=56 * 1024 * 1024),
    )(feat, fw[0], fb[0], fw[1], fb[1], fw[2], fb[2], fw[3], fb[3])
    return out
